# baseline (speedup 1.0000x reference)
import functools

import jax
import jax.numpy as jnp
from jax import lax
from jax.experimental import pallas as pl
from jax.experimental.pallas import tpu as pltpu
from jax.experimental.pallas import tpu_sc as plsc

_ROWS = 64
_N = 8192
_LANES = 16
_VREGS = _N // _LANES
_NC = 2
_NS = 16
_NW = _NC * _NS
_ROWS_PER_W = _ROWS // _NW
_PASSES = 4
_RADIX = 256
_CHUNKS = 32
_CVREGS = _VREGS // _CHUNKS
_CELEMS = _N // _CHUNKS
_HWORDS = _CHUNKS * _RADIX
_MIN32 = jnp.int32(-0x80000000)


def _sc_sort_rows(xm):
    mesh = plsc.VectorSubcoreMesh(core_axis_name="c", subcore_axis_name="s")

    vmem_k = pltpu.VMEM((_N,), jnp.float32)
    vmem_i = pltpu.VMEM((_N,), jnp.int32)
    vmem_h = pltpu.VMEM((_HWORDS,), jnp.int32)
    vmem_t = pltpu.VMEM((_RADIX,), jnp.int32)

    @functools.partial(
        pl.kernel,
        out_type=[
            jax.ShapeDtypeStruct((_ROWS, _N), jnp.float32),
            jax.ShapeDtypeStruct((_ROWS, _N), jnp.int32),
        ],
        mesh=mesh,
        compiler_params=pltpu.CompilerParams(needs_layout_passes=False),
        scratch_types=[
            vmem_k, vmem_k, vmem_i, vmem_i,
            vmem_k, vmem_k, vmem_i, vmem_i,
            vmem_h, vmem_h,
            vmem_h, vmem_h,
            vmem_t, vmem_t,
        ],
    )
    def sort_kernel(x_hbm, vals_hbm, idx_hbm,
                    kaA, kbA, iaA, ibA, kaB, kbB, iaB, ibB,
                    h0A, h1A, h0B, h1B, t0A, t0B):
        wid = lax.axis_index("s") * _NC + lax.axis_index("c")
        lane_iota = lax.iota(jnp.int32, _LANES)
        zeros16 = jnp.zeros((_LANES,), jnp.int32)
        row_a = wid * _ROWS_PER_W
        row_b = row_a + 1

        pltpu.sync_copy(x_hbm.at[row_a], kaA)
        pltpu.sync_copy(x_hbm.at[row_b], kaB)

        def zero_hists(ha, hb):
            def z_body(j):
                sl = pl.ds(j * _LANES, _LANES)
                ha[sl] = zeros16
                hb[sl] = zeros16
                return None

            plsc.parallel_loop(0, _HWORDS // _LANES, 1, unroll=4)(z_body)

        def hists_to_bases(ha, hb, ta, tb):
            ngroups = _RADIX // _LANES

            def c_body(c, accs):
                new_accs = []
                for g in range(ngroups):
                    sl = pl.ds(c * _RADIX + g * _LANES, _LANES)
                    va = ha[sl]
                    vb = hb[sl]
                    ha[sl] = accs[2 * g]
                    hb[sl] = accs[2 * g + 1]
                    new_accs.append(accs[2 * g] + va)
                    new_accs.append(accs[2 * g + 1] + vb)
                return tuple(new_accs)

            init = tuple(zeros16 for _ in range(2 * ngroups))
            accs = lax.fori_loop(0, _CHUNKS, c_body, init)

            sa = jnp.int32(0)
            sb = jnp.int32(0)
            for g in range(ngroups):
                va, vb = accs[2 * g], accs[2 * g + 1]
                inca = plsc.cumsum(va)
                incb = plsc.cumsum(vb)
                gsl = pl.ds(g * _LANES, _LANES)
                ta[gsl] = inca - va + sa
                tb[gsl] = incb - vb + sb
                sa = sa + jnp.squeeze(
                    lax.slice(inca, (_LANES - 1,), (_LANES,)))
                sb = sb + jnp.squeeze(
                    lax.slice(incb, (_LANES - 1,), (_LANES,)))

            def add_body(j):
                c = j // ngroups
                g = j % ngroups
                sl = pl.ds(c * _RADIX + g * _LANES, _LANES)
                gsl = pl.ds(g * _LANES, _LANES)
                ha[sl] = ha[sl] + ta[gsl]
                hb[sl] = hb[sl] + tb[gsl]
                return None

            plsc.parallel_loop(0, _CHUNKS * ngroups, 1, unroll=4)(add_body)

        zero_hists(h0A, h0B)

        def pro_body(i):
            sl = pl.ds(i * _LANES, _LANES)
            cbase = (i // _CVREGS) * _RADIX
            for ka, h0 in ((kaA, h0A), (kaB, h0B)):
                b = plsc.bitcast(ka[sl], jnp.int32)
                u = b ^ ((b >> 31) | _MIN32)
                ka[sl] = plsc.bitcast(u, jnp.float32)
                d = u & (_RADIX - 1)
                cnt, last_m = plsc.scan_count(cbase + d)
                plsc.addupdate_scatter(h0, [cbase + d], cnt, mask=last_m)
            return None

        plsc.parallel_loop(0, _VREGS, 1, unroll=4)(pro_body)

        bufsA = [(kaA, iaA), (kbA, ibA)]
        bufsB = [(kaB, iaB), (kbB, ibB)]
        histsA, histsB = [h0A, h1A], [h0B, h1B]
        for p in range(_PASSES):
            shift = p * 8
            last_pass = p == _PASSES - 1
            first_pass = p == 0
            hA, hB = histsA[p % 2], histsB[p % 2]
            hA_nxt, hB_nxt = histsA[(p + 1) % 2], histsB[(p + 1) % 2]

            hists_to_bases(hA, hB, t0A, t0B)
            if not last_pass:
                zero_hists(hA_nxt, hB_nxt)

            def step_body(j, _, p=p, shift=shift, last_pass=last_pass,
                          first_pass=first_pass, hA=hA, hB=hB,
                          hA_nxt=hA_nxt, hB_nxt=hB_nxt):

                def chunk_body(c):
                    cbase = c * _RADIX
                    i = c * _CVREGS + j
                    sl = pl.ds(i * _LANES, _LANES)
                    rows = (
                        (bufsA[p % 2], bufsA[(p + 1) % 2], hA, hA_nxt),
                        (bufsB[p % 2], bufsB[(p + 1) % 2], hB, hB_nxt),
                    )
                    for (k_in, i_in), (k_out, i_out), hist, hist_nxt in rows:
                        u = plsc.bitcast(k_in[sl], jnp.int32)
                        if first_pass:
                            ix = i * _LANES + lane_iota
                        else:
                            ix = i_in[sl]
                        d = lax.shift_right_logical(u, shift) & (_RADIX - 1)
                        cnt, last_m = plsc.scan_count(d)
                        base = plsc.load_gather(hist, [cbase + d])
                        dest = base + cnt - 1
                        if last_pass:
                            out_bits = u ^ (jnp.invert(u >> 31) | _MIN32)
                            store = plsc.bitcast(out_bits, jnp.float32)
                        else:
                            store = plsc.bitcast(u, jnp.float32)
                        plsc.store_scatter(k_out, [dest], store)
                        plsc.store_scatter(i_out, [dest], ix)
                        plsc.addupdate_scatter(hist, [cbase + d], cnt,
                                               mask=last_m)
                        if not last_pass:
                            d2 = lax.shift_right_logical(
                                u, shift + 8) & (_RADIX - 1)
                            cd2 = (lax.shift_right_logical(
                                dest, _CELEMS.bit_length() - 1)
                                   * _RADIX + d2)
                            cnt2, last2 = plsc.scan_count(cd2)
                            plsc.addupdate_scatter(hist_nxt, [cd2], cnt2,
                                                   mask=last2)
                    return None

                plsc.parallel_loop(0, _CHUNKS, 1, unroll=4)(chunk_body)
                return 0

            lax.fori_loop(0, _CVREGS, step_body, 0)

        pltpu.sync_copy(kaA, vals_hbm.at[row_a])
        pltpu.sync_copy(iaA, idx_hbm.at[row_a])
        pltpu.sync_copy(kaB, vals_hbm.at[row_b])
        pltpu.sync_copy(iaB, idx_hbm.at[row_b])

    return sort_kernel(xm)


def kernel(x, stable, dim, descending, values, indices):
    del stable, dim, values, indices
    desc = jnp.asarray(descending)
    xm = jnp.where(desc, -x, x)
    vals_m, idx = _sc_sort_rows(xm)
    vals = jnp.where(desc, -vals_m, vals_m)
    return vals, idx.astype(jnp.int64)


# speedup vs baseline: 1.0363x; 1.0363x over previous
import functools

import jax
import jax.numpy as jnp
from jax import lax
from jax.experimental import pallas as pl
from jax.experimental.pallas import tpu as pltpu
from jax.experimental.pallas import tpu_sc as plsc

_ROWS = 64
_N = 8192
_LANES = 16
_VREGS = _N // _LANES
_NC = 2
_NS = 16
_NW = _NC * _NS
_ROWS_PER_W = _ROWS // _NW
_PASSES = 4
_RADIX = 256
_CHUNKS = 16
_CVREGS = _VREGS // _CHUNKS
_CELEMS = _N // _CHUNKS
_HWORDS = _CHUNKS * _RADIX
_MIN32 = jnp.int32(-0x80000000)


def _sc_sort_rows(xm):
    mesh = plsc.VectorSubcoreMesh(core_axis_name="c", subcore_axis_name="s")

    vmem_k = pltpu.VMEM((_N,), jnp.float32)
    vmem_i = pltpu.VMEM((_N,), jnp.int32)
    vmem_h = pltpu.VMEM((_HWORDS,), jnp.int32)
    vmem_t = pltpu.VMEM((_RADIX,), jnp.int32)

    @functools.partial(
        pl.kernel,
        out_type=[
            jax.ShapeDtypeStruct((_ROWS, _N), jnp.float32),
            jax.ShapeDtypeStruct((_ROWS, _N), jnp.int32),
        ],
        mesh=mesh,
        compiler_params=pltpu.CompilerParams(needs_layout_passes=False),
        scratch_types=[
            vmem_k, vmem_k, vmem_i, vmem_i,
            vmem_k, vmem_k, vmem_i, vmem_i,
            vmem_h, vmem_h,
            vmem_h, vmem_h,
            vmem_t, vmem_t,
        ],
    )
    def sort_kernel(x_hbm, vals_hbm, idx_hbm,
                    kaA, kbA, iaA, ibA, kaB, kbB, iaB, ibB,
                    h0A, h1A, h0B, h1B, t0A, t0B):
        wid = lax.axis_index("s") * _NC + lax.axis_index("c")
        lane_iota = lax.iota(jnp.int32, _LANES)
        zeros16 = jnp.zeros((_LANES,), jnp.int32)
        row_a = wid * _ROWS_PER_W
        row_b = row_a + 1

        pltpu.sync_copy(x_hbm.at[row_a], kaA)
        pltpu.sync_copy(x_hbm.at[row_b], kaB)

        def zero_hists(ha, hb):
            def z_body(j):
                sl = pl.ds(j * _LANES, _LANES)
                ha[sl] = zeros16
                hb[sl] = zeros16
                return None

            plsc.parallel_loop(0, _HWORDS // _LANES, 1, unroll=4)(z_body)

        def hists_to_bases(ha, hb, ta, tb):
            ngroups = _RADIX // _LANES

            def c_body(c, accs):
                new_accs = []
                for g in range(ngroups):
                    sl = pl.ds(c * _RADIX + g * _LANES, _LANES)
                    va = ha[sl]
                    vb = hb[sl]
                    ha[sl] = accs[2 * g]
                    hb[sl] = accs[2 * g + 1]
                    new_accs.append(accs[2 * g] + va)
                    new_accs.append(accs[2 * g + 1] + vb)
                return tuple(new_accs)

            init = tuple(zeros16 for _ in range(2 * ngroups))
            accs = lax.fori_loop(0, _CHUNKS, c_body, init)

            sa = jnp.int32(0)
            sb = jnp.int32(0)
            for g in range(ngroups):
                va, vb = accs[2 * g], accs[2 * g + 1]
                inca = plsc.cumsum(va)
                incb = plsc.cumsum(vb)
                gsl = pl.ds(g * _LANES, _LANES)
                ta[gsl] = inca - va + sa
                tb[gsl] = incb - vb + sb
                sa = sa + jnp.squeeze(
                    lax.slice(inca, (_LANES - 1,), (_LANES,)))
                sb = sb + jnp.squeeze(
                    lax.slice(incb, (_LANES - 1,), (_LANES,)))

            def add_body(j):
                c = j // ngroups
                g = j % ngroups
                sl = pl.ds(c * _RADIX + g * _LANES, _LANES)
                gsl = pl.ds(g * _LANES, _LANES)
                ha[sl] = ha[sl] + ta[gsl]
                hb[sl] = hb[sl] + tb[gsl]
                return None

            plsc.parallel_loop(0, _CHUNKS * ngroups, 1, unroll=4)(add_body)

        zero_hists(h0A, h0B)

        def pro_body(i):
            sl = pl.ds(i * _LANES, _LANES)
            cbase = (i // _CVREGS) * _RADIX
            for ka, h0 in ((kaA, h0A), (kaB, h0B)):
                b = plsc.bitcast(ka[sl], jnp.int32)
                u = b ^ ((b >> 31) | _MIN32)
                ka[sl] = plsc.bitcast(u, jnp.float32)
                d = u & (_RADIX - 1)
                cnt, last_m = plsc.scan_count(cbase + d)
                plsc.addupdate_scatter(h0, [cbase + d], cnt, mask=last_m)
            return None

        plsc.parallel_loop(0, _VREGS, 1, unroll=4)(pro_body)

        bufsA = [(kaA, iaA), (kbA, ibA)]
        bufsB = [(kaB, iaB), (kbB, ibB)]
        histsA, histsB = [h0A, h1A], [h0B, h1B]
        for p in range(_PASSES):
            shift = p * 8
            last_pass = p == _PASSES - 1
            first_pass = p == 0
            hA, hB = histsA[p % 2], histsB[p % 2]
            hA_nxt, hB_nxt = histsA[(p + 1) % 2], histsB[(p + 1) % 2]

            hists_to_bases(hA, hB, t0A, t0B)
            if not last_pass:
                zero_hists(hA_nxt, hB_nxt)

            def step_body(j, _, p=p, shift=shift, last_pass=last_pass,
                          first_pass=first_pass, hA=hA, hB=hB,
                          hA_nxt=hA_nxt, hB_nxt=hB_nxt):

                def chunk_body(c):
                    cbase = c * _RADIX
                    i = c * _CVREGS + j
                    sl = pl.ds(i * _LANES, _LANES)
                    rows = (
                        (bufsA[p % 2], bufsA[(p + 1) % 2], hA, hA_nxt),
                        (bufsB[p % 2], bufsB[(p + 1) % 2], hB, hB_nxt),
                    )
                    for (k_in, i_in), (k_out, i_out), hist, hist_nxt in rows:
                        u = plsc.bitcast(k_in[sl], jnp.int32)
                        if first_pass:
                            ix = i * _LANES + lane_iota
                        else:
                            ix = i_in[sl]
                        d = lax.shift_right_logical(u, shift) & (_RADIX - 1)
                        cnt, last_m = plsc.scan_count(d)
                        base = plsc.load_gather(hist, [cbase + d])
                        dest = base + cnt - 1
                        if last_pass:
                            out_bits = u ^ (jnp.invert(u >> 31) | _MIN32)
                            store = plsc.bitcast(out_bits, jnp.float32)
                        else:
                            store = plsc.bitcast(u, jnp.float32)
                        plsc.store_scatter(k_out, [dest], store)
                        plsc.store_scatter(i_out, [dest], ix)
                        plsc.addupdate_scatter(hist, [cbase + d], cnt,
                                               mask=last_m)
                        if not last_pass:
                            d2 = lax.shift_right_logical(
                                u, shift + 8) & (_RADIX - 1)
                            cd2 = (lax.shift_right_logical(dest, 9)
                                   * _RADIX + d2)
                            cnt2, last2 = plsc.scan_count(cd2)
                            plsc.addupdate_scatter(hist_nxt, [cd2], cnt2,
                                                   mask=last2)
                    return None

                plsc.parallel_loop(0, _CHUNKS, 1, unroll=4)(chunk_body)
                return 0

            lax.fori_loop(0, _CVREGS, step_body, 0)

        pltpu.sync_copy(kaA, vals_hbm.at[row_a])
        pltpu.sync_copy(iaA, idx_hbm.at[row_a])
        pltpu.sync_copy(kaB, vals_hbm.at[row_b])
        pltpu.sync_copy(iaB, idx_hbm.at[row_b])

    return sort_kernel(xm)


def kernel(x, stable, dim, descending, values, indices):
    del stable, dim, values, indices
    desc = jnp.asarray(descending)
    xm = jnp.where(desc, -x, x)
    vals_m, idx = _sc_sort_rows(xm)
    vals = jnp.where(desc, -vals_m, vals_m)
    return vals, idx.astype(jnp.int64)
